# baseline (speedup 1.0000x reference)
import functools

import jax
import jax.numpy as jnp
from jax import lax
from jax.experimental import pallas as pl
from jax.experimental.pallas import tpu as pltpu
from jax.experimental.pallas import tpu_sc as plsc

_LAMADA = 0.001
_B = 16384
_D = 64
_NC = 2
_NS = 16
_NW = _NC * _NS
_BPW = _B // _NW
_CHUNK = 128
_NCHUNK = _BPW // _CHUNK
_UNROLL = 16

_PARAMS = pltpu.CompilerParams(
    needs_layout_passes=False, use_tc_tiling_on_sc=False)
_MESH = dict(core_axis_name="c", subcore_axis_name="s")


def _wid():
    return lax.axis_index("s") * _NC + lax.axis_index("c")


def _gather_body(idx_hbm, table_hbm, sel_hbm, psum_hbm, idxv, rows, res, sem):
    wid = _wid()
    base = wid * _BPW
    pltpu.sync_copy(idx_hbm.at[pl.ds(base, _BPW)], idxv)
    copies = []
    for j in range(_NCHUNK):
        sl = pl.ds(j * _CHUNK, _CHUNK)
        copies.append(pltpu.async_copy(table_hbm.at[idxv.at[sl]], rows.at[sl], sem))
    for c in copies:
        c.wait()

    zeros = jnp.zeros((16,), jnp.float32)

    def row_block(blk, acc):
        for k in range(_UNROLL):
            r = blk * _UNROLL + k
            for c in range(_D // 32):
                a, b = plsc.unpack(rows[r, pl.ds(32 * c, 32)],
                                   format=plsc.PackFormat.INTERLEAVED)
                acc = acc + a * a + b * b
        return acc

    sq = lax.fori_loop(0, _BPW // _UNROLL, row_block, zeros)
    sq_s = jnp.sum(sq) * (_LAMADA / (_B * _D))

    pltpu.sync_copy(rows, sel_hbm.at[pl.ds(base, _BPW), :])

    lane = lax.iota(jnp.int32, 16)
    res[...] = jnp.where(lane == 0, jnp.full((16,), sq_s), zeros)
    pltpu.sync_copy(res, psum_hbm.at[wid])


def _gather_dot_body(idx_hbm, ratings_hbm, table_hbm, usel_hbm, psum_hbm,
                     idxv, irows, urows, rat, res, sem):
    wid = _wid()
    base = wid * _BPW
    pltpu.sync_copy(idx_hbm.at[pl.ds(base, _BPW)], idxv)
    copies = [pltpu.async_copy(usel_hbm.at[pl.ds(base, _BPW), :], urows, sem)]
    for j in range(_NCHUNK):
        sl = pl.ds(j * _CHUNK, _CHUNK)
        copies.append(pltpu.async_copy(table_hbm.at[idxv.at[sl]], irows.at[sl], sem))
    pltpu.sync_copy(ratings_hbm.at[pl.ds(base, _BPW)], rat)
    for c in copies:
        c.wait()

    zeros = jnp.zeros((16,), jnp.float32)

    def row_block(blk, carry):
        loss2_acc, i2_acc = carry
        rv = rat[pl.ds(blk * _UNROLL, 16)]
        for k in range(_UNROLL):
            r = blk * _UNROLL + k
            t = None
            for c in range(_D // 32):
                ua, ub = plsc.unpack(urows[r, pl.ds(32 * c, 32)],
                                     format=plsc.PackFormat.INTERLEAVED)
                va, vb = plsc.unpack(irows[r, pl.ds(32 * c, 32)],
                                     format=plsc.PackFormat.INTERLEAVED)
                p = ua * va + ub * vb
                t = p if t is None else t + p
                i2_acc = i2_acc + va * va + vb * vb
            err = jnp.sum(t) - rv[k]
            loss2_acc = loss2_acc + err * err
        return (loss2_acc, i2_acc)

    loss2_s, i2_acc = lax.fori_loop(
        0, _BPW // _UNROLL, row_block, (jnp.float32(0.0), zeros))
    loss2_s = loss2_s * (1.0 / _B)
    i2_s = jnp.sum(i2_acc) * (_LAMADA / (_B * _D))

    lane = lax.iota(jnp.int32, 16)
    res[...] = (jnp.where(lane == 0, jnp.full((16,), loss2_s), zeros)
                + jnp.where(lane == 1, jnp.full((16,), i2_s), zeros))
    pltpu.sync_copy(res, psum_hbm.at[wid])


def _gather_call(idx, table):
    kfn = functools.partial(
        pl.kernel,
        out_type=(jax.ShapeDtypeStruct((_B, _D), jnp.bfloat16),
                  jax.ShapeDtypeStruct((_NW, 16), jnp.float32)),
        mesh=plsc.VectorSubcoreMesh(**_MESH),
        compiler_params=_PARAMS,
        scratch_types=[
            pltpu.VMEM((_BPW,), jnp.int32),
            pltpu.VMEM((_BPW, _D), jnp.bfloat16),
            pltpu.VMEM((16,), jnp.float32),
            pltpu.SemaphoreType.DMA,
        ],
    )(_gather_body)
    return kfn(idx, table)


def _gather_dot_call(item_i0, ratings, embed_item, usel):
    kfn = functools.partial(
        pl.kernel,
        out_type=jax.ShapeDtypeStruct((_NW, 16), jnp.float32),
        mesh=plsc.VectorSubcoreMesh(**_MESH),
        compiler_params=_PARAMS,
        scratch_types=[
            pltpu.VMEM((_BPW,), jnp.int32),
            pltpu.VMEM((_BPW, _D), jnp.bfloat16),
            pltpu.VMEM((_BPW, _D), jnp.bfloat16),
            pltpu.VMEM((_BPW,), jnp.float32),
            pltpu.VMEM((16,), jnp.float32),
            pltpu.SemaphoreType.DMA,
        ],
    )(_gather_dot_body)
    return kfn(item_i0, ratings, embed_item, usel)


def kernel(user0, item_i0, ratings, embed_user, embed_item):
    usel, up = _gather_call(user0, embed_user.astype(jnp.bfloat16))
    dp = _gather_dot_call(item_i0, ratings,
                          embed_item.astype(jnp.bfloat16), usel)
    loss2 = jnp.sum(dp[:, 0])
    l2 = jnp.sum(up[:, 0]) + jnp.sum(dp[:, 1])
    return (loss2 + l2, loss2, l2)


# speedup vs baseline: 1.3794x; 1.3794x over previous
import functools

import jax
import jax.numpy as jnp
from jax import lax
from jax.experimental import pallas as pl
from jax.experimental.pallas import tpu as pltpu
from jax.experimental.pallas import tpu_sc as plsc

_LAMADA = 0.001
_B = 16384
_D = 64
_NC = 2
_NS = 16
_NW = _NC * _NS
_BPW = _B // _NW
_CHUNK = 128
_NCHUNK = _BPW // _CHUNK
_UNROLL = 16

_PARAMS = pltpu.CompilerParams(
    needs_layout_passes=False, use_tc_tiling_on_sc=False)
_MESH = dict(core_axis_name="c", subcore_axis_name="s")


def _wid():
    return lax.axis_index("s") * _NC + lax.axis_index("c")


def _gather_body(idx_hbm, table_hbm, sel_hbm, psum_hbm, idxv, rows, res, sem):
    wid = _wid()
    base = wid * _BPW
    pltpu.sync_copy(idx_hbm.at[pl.ds(base, _BPW)], idxv)
    copies = []
    for j in range(_NCHUNK):
        sl = pl.ds(j * _CHUNK, _CHUNK)
        copies.append(pltpu.async_copy(table_hbm.at[idxv.at[sl]], rows.at[sl], sem))
    for c in copies:
        c.wait()

    zeros = jnp.zeros((16,), jnp.float32)

    def row_block(blk, acc):
        for k in range(_UNROLL):
            r = blk * _UNROLL + k
            for c in range(_D // 16):
                v = rows[r, pl.ds(16 * c, 16)]
                acc = acc + v * v
        return acc

    sq = lax.fori_loop(0, _BPW // _UNROLL, row_block, zeros)
    sq_s = jnp.sum(sq) * (_LAMADA / (_B * _D))

    pltpu.sync_copy(rows, sel_hbm.at[pl.ds(base, _BPW), :])

    lane = lax.iota(jnp.int32, 16)
    res[...] = jnp.where(lane == 0, jnp.full((16,), sq_s), zeros)
    pltpu.sync_copy(res, psum_hbm.at[wid])


def _gather_dot_body(idx_hbm, ratings_hbm, table_hbm, usel_hbm, psum_hbm,
                     idxv, irows, urows, rat, res, sem):
    wid = _wid()
    base = wid * _BPW
    pltpu.sync_copy(idx_hbm.at[pl.ds(base, _BPW)], idxv)
    copies = [pltpu.async_copy(usel_hbm.at[pl.ds(base, _BPW), :], urows, sem)]
    for j in range(_NCHUNK):
        sl = pl.ds(j * _CHUNK, _CHUNK)
        copies.append(pltpu.async_copy(table_hbm.at[idxv.at[sl]], irows.at[sl], sem))
    pltpu.sync_copy(ratings_hbm.at[pl.ds(base, _BPW)], rat)
    for c in copies:
        c.wait()

    zeros = jnp.zeros((16,), jnp.float32)

    def row_block(blk, carry):
        loss2_acc, i2_acc = carry
        rv = rat[pl.ds(blk * _UNROLL, 16)]
        for k in range(_UNROLL):
            r = blk * _UNROLL + k
            us = [urows[r, pl.ds(16 * c, 16)] for c in range(_D // 16)]
            vs = [irows[r, pl.ds(16 * c, 16)] for c in range(_D // 16)]
            t = us[0] * vs[0]
            for c in range(1, _D // 16):
                t = t + us[c] * vs[c]
            err = jnp.sum(t) - rv[k]
            loss2_acc = loss2_acc + err * err
            for c in range(_D // 16):
                i2_acc = i2_acc + vs[c] * vs[c]
        return (loss2_acc, i2_acc)

    loss2_s, i2_acc = lax.fori_loop(
        0, _BPW // _UNROLL, row_block, (jnp.float32(0.0), zeros))
    loss2_s = loss2_s * (1.0 / _B)
    i2_s = jnp.sum(i2_acc) * (_LAMADA / (_B * _D))

    lane = lax.iota(jnp.int32, 16)
    res[...] = (jnp.where(lane == 0, jnp.full((16,), loss2_s), zeros)
                + jnp.where(lane == 1, jnp.full((16,), i2_s), zeros))
    pltpu.sync_copy(res, psum_hbm.at[wid])


def _gather_call(idx, table):
    kfn = functools.partial(
        pl.kernel,
        out_type=(jax.ShapeDtypeStruct((_B, _D), jnp.float32),
                  jax.ShapeDtypeStruct((_NW, 16), jnp.float32)),
        mesh=plsc.VectorSubcoreMesh(**_MESH),
        compiler_params=_PARAMS,
        scratch_types=[
            pltpu.VMEM((_BPW,), jnp.int32),
            pltpu.VMEM((_BPW, _D), jnp.float32),
            pltpu.VMEM((16,), jnp.float32),
            pltpu.SemaphoreType.DMA,
        ],
    )(_gather_body)
    return kfn(idx, table)


def _gather_dot_call(item_i0, ratings, embed_item, usel):
    kfn = functools.partial(
        pl.kernel,
        out_type=jax.ShapeDtypeStruct((_NW, 16), jnp.float32),
        mesh=plsc.VectorSubcoreMesh(**_MESH),
        compiler_params=_PARAMS,
        scratch_types=[
            pltpu.VMEM((_BPW,), jnp.int32),
            pltpu.VMEM((_BPW, _D), jnp.float32),
            pltpu.VMEM((_BPW, _D), jnp.float32),
            pltpu.VMEM((_BPW,), jnp.float32),
            pltpu.VMEM((16,), jnp.float32),
            pltpu.SemaphoreType.DMA,
        ],
    )(_gather_dot_body)
    return kfn(item_i0, ratings, embed_item, usel)


def kernel(user0, item_i0, ratings, embed_user, embed_item):
    usel, up = _gather_call(user0, embed_user)
    dp = _gather_dot_call(item_i0, ratings, embed_item, usel)
    loss2 = jnp.sum(dp[:, 0])
    l2 = jnp.sum(up[:, 0]) + jnp.sum(dp[:, 1])
    return (loss2 + l2, loss2, l2)
